# baseline (speedup 1.0000x reference)
import jax
import jax.numpy as jnp
from jax.experimental import pallas as pl
from jax.experimental.pallas import tpu as pltpu

B, N, M = 4, 4096, 4096
BLK_N = 256
NB = N // BLK_N


def _chamfer_body(pred_ref, gt_ref, out_ref, dist2_ref, acc_ref):
    b = pl.program_id(0)
    i = pl.program_id(1)

    p = pred_ref[0]
    g = gt_ref[0]

    px = p[:, 0:1]
    py = p[:, 1:2]
    pz = p[:, 2:3]
    gx = g[0:1, :]
    gy = g[1:2, :]
    gz = g[2:3, :]

    dx = px - gx
    dy = py - gy
    dz = pz - gz
    d = dx * dx + dy * dy + dz * dz

    rowmin = jnp.min(d, axis=1)
    colmin = jnp.min(d, axis=0, keepdims=True)

    @pl.when(i == 0)
    def _():
        dist2_ref[...] = colmin

    @pl.when(i > 0)
    def _():
        dist2_ref[...] = jnp.minimum(dist2_ref[...], colmin)

    bsum = jnp.sum(rowmin)
    bmax = jnp.max(rowmin)

    @pl.when(i == 0)
    def _():
        acc_ref[0] = bsum
        acc_ref[1] = bmax

    @pl.when(i > 0)
    def _():
        acc_ref[0] = acc_ref[0] + bsum
        acc_ref[1] = jnp.maximum(acc_ref[1], bmax)

    @pl.when(jnp.logical_and(b == 0, i == 0))
    def _():
        out_ref[0, 0] = 0.0

    @pl.when(i == NB - 1)
    def _():
        mean1 = acc_ref[0] / N
        max1 = acc_ref[1]
        mean2 = jnp.sum(dist2_ref[...]) / M
        out_ref[0, 0] = out_ref[0, 0] + (mean1 + mean2 + max1) / B


def kernel(pred, gt):
    gt_t = jnp.transpose(gt, (0, 2, 1))

    out = pl.pallas_call(
        _chamfer_body,
        grid=(B, NB),
        in_specs=[
            pl.BlockSpec((1, BLK_N, 3), lambda b, i: (b, i, 0)),
            pl.BlockSpec((1, 3, M), lambda b, i: (b, 0, 0)),
        ],
        out_specs=pl.BlockSpec(
            (1, 1), lambda b, i: (0, 0), memory_space=pltpu.SMEM
        ),
        out_shape=jax.ShapeDtypeStruct((1, 1), jnp.float32),
        scratch_shapes=[
            pltpu.VMEM((1, M), jnp.float32),
            pltpu.SMEM((2,), jnp.float32),
        ],
    )(pred, gt_t)
    return out[0, 0]


# speedup vs baseline: 2.1901x; 2.1901x over previous
import jax
import jax.numpy as jnp
from jax.experimental import pallas as pl
from jax.experimental.pallas import tpu as pltpu

B, N, M = 4, 4096, 4096
BLK_N = 256
NB = N // BLK_N


def _chamfer_body(a_ref, g_ref, out_ref, dist2_ref, acc_ref):
    b = pl.program_id(0)
    i = pl.program_id(1)

    a = a_ref[0]
    g = g_ref[0]

    d = jax.lax.dot_general(
        a, g, (((1,), (0,)), ((), ())), preferred_element_type=jnp.float32
    )

    rowmin = jnp.min(d, axis=1)
    colmin = jnp.min(d, axis=0, keepdims=True)

    @pl.when(i == 0)
    def _():
        dist2_ref[...] = colmin

    @pl.when(i > 0)
    def _():
        dist2_ref[...] = jnp.minimum(dist2_ref[...], colmin)

    bsum = jnp.sum(rowmin)
    bmax = jnp.max(rowmin)

    @pl.when(i == 0)
    def _():
        acc_ref[0] = bsum
        acc_ref[1] = bmax

    @pl.when(i > 0)
    def _():
        acc_ref[0] = acc_ref[0] + bsum
        acc_ref[1] = jnp.maximum(acc_ref[1], bmax)

    @pl.when(jnp.logical_and(b == 0, i == 0))
    def _():
        out_ref[0, 0] = 0.0

    @pl.when(i == NB - 1)
    def _():
        mean1 = acc_ref[0] / N
        max1 = acc_ref[1]
        mean2 = jnp.sum(dist2_ref[...]) / M
        out_ref[0, 0] = out_ref[0, 0] + (mean1 + mean2 + max1) / B


def kernel(pred, gt):
    x2 = jnp.sum(pred * pred, axis=-1, keepdims=True)
    y2 = jnp.sum(gt * gt, axis=-1, keepdims=True)
    ones = jnp.ones_like(x2)
    zeros = jnp.zeros((B, N, 3), jnp.float32)
    a = jnp.concatenate([ones, x2, -2.0 * pred, zeros], axis=-1)
    gmat = jnp.concatenate([y2, ones, gt, zeros], axis=-1)
    gmat_t = jnp.transpose(gmat, (0, 2, 1))

    out = pl.pallas_call(
        _chamfer_body,
        grid=(B, NB),
        in_specs=[
            pl.BlockSpec((1, BLK_N, 8), lambda b, i: (b, i, 0)),
            pl.BlockSpec((1, 8, M), lambda b, i: (b, 0, 0)),
        ],
        out_specs=pl.BlockSpec(
            (1, 1), lambda b, i: (0, 0), memory_space=pltpu.SMEM
        ),
        out_shape=jax.ShapeDtypeStruct((1, 1), jnp.float32),
        scratch_shapes=[
            pltpu.VMEM((1, M), jnp.float32),
            pltpu.SMEM((2,), jnp.float32),
        ],
    )(a, gmat_t)
    return out[0, 0]


# speedup vs baseline: 2.5529x; 1.1657x over previous
import jax
import jax.numpy as jnp
from jax.experimental import pallas as pl
from jax.experimental.pallas import tpu as pltpu

B, N, M = 4, 4096, 4096
BLK_N = 512
NB = N // BLK_N
BLK_M = 1024
NC = M // BLK_M


def _chamfer_body(a_ref, g_ref, out_ref, dist2_ref, acc_ref):
    b = pl.program_id(0)
    i = pl.program_id(1)

    a = a_ref[0]

    rowmins = []
    colmins = []
    for j in range(NC):
        g = g_ref[0, :, j * BLK_M:(j + 1) * BLK_M]
        d = jax.lax.dot_general(
            a, g, (((1,), (0,)), ((), ())),
            preferred_element_type=jnp.float32,
        )
        rowmins.append(jnp.min(d, axis=1, keepdims=True))
        colmins.append(jnp.min(d, axis=0, keepdims=True))

    rowmin = jnp.min(jnp.concatenate(rowmins, axis=1), axis=1)
    colmin = jnp.concatenate(colmins, axis=1)

    @pl.when(i == 0)
    def _():
        dist2_ref[...] = colmin

    @pl.when(i > 0)
    def _():
        dist2_ref[...] = jnp.minimum(dist2_ref[...], colmin)

    bsum = jnp.sum(rowmin)
    bmax = jnp.max(rowmin)

    @pl.when(i == 0)
    def _():
        acc_ref[0] = bsum
        acc_ref[1] = bmax

    @pl.when(i > 0)
    def _():
        acc_ref[0] = acc_ref[0] + bsum
        acc_ref[1] = jnp.maximum(acc_ref[1], bmax)

    @pl.when(jnp.logical_and(b == 0, i == 0))
    def _():
        out_ref[0, 0] = 0.0

    @pl.when(i == NB - 1)
    def _():
        mean1 = acc_ref[0] / N
        max1 = acc_ref[1]
        mean2 = jnp.sum(dist2_ref[...]) / M
        out_ref[0, 0] = out_ref[0, 0] + (mean1 + mean2 + max1) / B


def kernel(pred, gt):
    x2 = jnp.sum(pred * pred, axis=-1, keepdims=True)
    y2 = jnp.sum(gt * gt, axis=-1, keepdims=True)
    ones = jnp.ones_like(x2)
    zeros = jnp.zeros((B, N, 3), jnp.float32)
    a = jnp.concatenate([ones, x2, -2.0 * pred, zeros], axis=-1)
    gmat = jnp.concatenate([y2, ones, gt, zeros], axis=-1)
    gmat_t = jnp.transpose(gmat, (0, 2, 1))

    out = pl.pallas_call(
        _chamfer_body,
        grid=(B, NB),
        in_specs=[
            pl.BlockSpec((1, BLK_N, 8), lambda b, i: (b, i, 0)),
            pl.BlockSpec((1, 8, M), lambda b, i: (b, 0, 0)),
        ],
        out_specs=pl.BlockSpec(
            (1, 1), lambda b, i: (0, 0), memory_space=pltpu.SMEM
        ),
        out_shape=jax.ShapeDtypeStruct((1, 1), jnp.float32),
        scratch_shapes=[
            pltpu.VMEM((1, M), jnp.float32),
            pltpu.SMEM((2,), jnp.float32),
        ],
    )(a, gmat_t)
    return out[0, 0]
